# baseline (speedup 1.0000x reference)
import jax
import jax.numpy as jnp
from jax.experimental import pallas as pl
from jax.experimental.pallas import tpu as pltpu

B = 4096
D_NUM = 256
NUM_HID = 256
TEXT_DIM = 768
FUSION = 1024
E = 8
N_TRAITS = 5
N_CHAR = 10

M_TILE = 1024
M_TILES = B // M_TILE
E_PER = 2
E_STEPS = E // E_PER


def _ln(x, g, b):
    mu = jnp.mean(x, axis=-1, keepdims=True)
    var = jnp.var(x, axis=-1, keepdims=True)
    return (x - mu) / jnp.sqrt(var + 1e-5) * g + b


def _moe_kernel(
    x_ref, Wn_ref, bn_ref, gn_ref, bln_ref, Wfn_ref, bf_ref, gf_ref, blf_ref,
    gr_ref, blr_ref, Wr1_ref, br1_ref, Wr2_ref, br2_ref,
    We1_ref, be1_ref, We2_ref, be2_ref,
    W1h_ref, b1h_ref, W2h_ref, b2h_ref,
    heads_ref, lb_ref, ent_ref,
    fused_s, w_s, refined_s, usage_s, ent_s,
):
    m = pl.program_id(0)
    ep = pl.program_id(1)

    @pl.when(ep == 0)
    def _prologue():
        x = x_ref[...]
        num = jax.nn.relu(_ln(jnp.dot(x, Wn_ref[...]) + bn_ref[...],
                              gn_ref[...], bln_ref[...]))
        fused = jax.nn.relu(_ln(jnp.dot(num, Wfn_ref[...]) + bf_ref[...],
                                gf_ref[...], blf_ref[...]))
        fused_s[...] = fused.astype(jnp.bfloat16)
        h = _ln(fused, gr_ref[...], blr_ref[...])
        h1 = jax.nn.relu(jnp.dot(h, Wr1_ref[...]) + br1_ref[...])
        logits = jnp.dot(h1, Wr2_ref[...]) + br2_ref[...]
        w = jax.nn.softmax(logits, axis=-1)
        w_s[...] = w

        @pl.when(m == 0)
        def _init_scalars():
            usage_s[...] = jnp.zeros_like(usage_s)
            ent_s[0, 0] = 0.0

        usage_s[...] += jnp.sum(w, axis=0, keepdims=True)
        ent_s[0, 0] += jnp.sum(w * jnp.log(w + 1e-12))
        refined_s[...] = jnp.dot(w, be2_ref[...])

    fused = fused_s[...].astype(jnp.float32)
    w = w_s[...]
    lane = jax.lax.broadcasted_iota(jnp.int32, (1, E), 1)
    acc = refined_s[...]
    for j in range(E_PER):
        e_abs = ep * E_PER + j
        eh = jax.nn.relu(jnp.dot(fused, We1_ref[j]) + be1_ref[j])
        col = jnp.sum(jnp.where(lane == e_abs, w, 0.0), axis=-1, keepdims=True)
        acc = acc + jnp.dot(eh * col, We2_ref[j])
    refined_s[...] = acc

    @pl.when(ep == E_STEPS - 1)
    def _epilogue():
        r = refined_s[...]
        h1h = jax.nn.relu(jnp.dot(r, W1h_ref[...]) + b1h_ref[...])
        heads_ref[...] = jnp.dot(h1h, W2h_ref[...]) + b2h_ref[...]

        @pl.when(m == M_TILES - 1)
        def _scalars():
            mu = usage_s[...] / B
            lb = jnp.mean((mu - 1.0 / E) ** 2)
            lb_ref[...] = jnp.full((1, 1), lb, jnp.float32)
            ent_ref[...] = jnp.full((1, 1), -ent_s[0, 0] / B, jnp.float32)


def kernel(numeric_features, Wn, bn, gn, bln, Wf, bf, gf, blf, gr, blr,
           Wr1, br1, Wr2, br2, We1, be1, We2, be2, Wt1, bt1, Wt2, bt2,
           Wi1, bi1, Wi2, bi2, Wc1, bc1, Wc2, bc2):
    Wfn = Wf[TEXT_DIM:, :]
    row = lambda v: v.reshape(1, -1)
    N_HEADS = N_TRAITS + 3 + N_CHAR
    W1h = jnp.concatenate([Wt1, Wi1, Wc1], axis=1)
    b1h = jnp.concatenate([bt1, bi1, bc1])
    W2h = jnp.zeros((3 * 256, N_HEADS), jnp.float32)
    W2h = W2h.at[0:256, 0:N_TRAITS].set(Wt2)
    W2h = W2h.at[256:512, N_TRAITS:N_TRAITS + 3].set(Wi2)
    W2h = W2h.at[512:768, N_TRAITS + 3:N_HEADS].set(Wc2)
    b2h = jnp.concatenate([bt2, bi2, bc2])

    const = lambda *_: (0, 0)
    by_m = lambda m, ep: (m, 0)
    by_e3 = lambda m, ep: (ep, 0, 0)

    grid = (M_TILES, E_STEPS)
    out = pl.pallas_call(
        _moe_kernel,
        grid=grid,
        in_specs=[
            pl.BlockSpec((M_TILE, D_NUM), by_m),
            pl.BlockSpec((D_NUM, NUM_HID), const),
            pl.BlockSpec((1, NUM_HID), const),
            pl.BlockSpec((1, NUM_HID), const),
            pl.BlockSpec((1, NUM_HID), const),
            pl.BlockSpec((NUM_HID, FUSION), const),
            pl.BlockSpec((1, FUSION), const),
            pl.BlockSpec((1, FUSION), const),
            pl.BlockSpec((1, FUSION), const),
            pl.BlockSpec((1, FUSION), const),
            pl.BlockSpec((1, FUSION), const),
            pl.BlockSpec((FUSION, FUSION // 2), const),
            pl.BlockSpec((1, FUSION // 2), const),
            pl.BlockSpec((FUSION // 2, E), const),
            pl.BlockSpec((1, E), const),
            pl.BlockSpec((E_PER, FUSION, FUSION), by_e3),
            pl.BlockSpec((E_PER, 1, FUSION), by_e3),
            pl.BlockSpec((E_PER, FUSION, FUSION), by_e3),
            pl.BlockSpec((E, FUSION), const),
            pl.BlockSpec((FUSION, 3 * 256), const),
            pl.BlockSpec((1, 3 * 256), const),
            pl.BlockSpec((3 * 256, N_HEADS), const),
            pl.BlockSpec((1, N_HEADS), const),
        ],
        out_specs=[
            pl.BlockSpec((M_TILE, N_HEADS), by_m),
            pl.BlockSpec((1, 1), const),
            pl.BlockSpec((1, 1), const),
        ],
        out_shape=[
            jax.ShapeDtypeStruct((B, N_HEADS), jnp.float32),
            jax.ShapeDtypeStruct((1, 1), jnp.float32),
            jax.ShapeDtypeStruct((1, 1), jnp.float32),
        ],
        scratch_shapes=[
            pltpu.VMEM((M_TILE, FUSION), jnp.bfloat16),
            pltpu.VMEM((M_TILE, E), jnp.float32),
            pltpu.VMEM((M_TILE, FUSION), jnp.float32),
            pltpu.VMEM((1, E), jnp.float32),
            pltpu.SMEM((1, 1), jnp.float32),
        ],
        compiler_params=pltpu.CompilerParams(
            dimension_semantics=("parallel", "arbitrary"),
            vmem_limit_bytes=63 * 1024 * 1024,
        ),
    )(
        numeric_features, Wn, row(bn), row(gn), row(bln), Wfn, row(bf),
        row(gf), row(blf), row(gr), row(blr), Wr1, row(br1), Wr2, row(br2),
        We1, be1.reshape(E, 1, FUSION), We2, be2,
        W1h, row(b1h), W2h, row(b2h),
    )
    heads, lb, ent = out
    trait = heads[:, :N_TRAITS]
    irt = heads[:, N_TRAITS:N_TRAITS + 3]
    char = heads[:, N_TRAITS + 3:]
    return trait, irt, char, lb.reshape(()), ent.reshape(())


# speedup vs baseline: 1.0015x; 1.0015x over previous
import jax
import jax.numpy as jnp
from jax.experimental import pallas as pl
from jax.experimental.pallas import tpu as pltpu

B = 4096
D_NUM = 256
NUM_HID = 256
TEXT_DIM = 768
FUSION = 1024
E = 8
N_TRAITS = 5
N_CHAR = 10

M_TILE = 1024
M_TILES = B // M_TILE
E_PER = 2
E_STEPS = E // E_PER


def _ln(x, g, b):
    mu = jnp.mean(x, axis=-1, keepdims=True)
    var = jnp.var(x, axis=-1, keepdims=True)
    return (x - mu) / jnp.sqrt(var + 1e-5) * g + b


def _moe_kernel(
    x_ref, Wn_ref, bn_ref, gn_ref, bln_ref, Wfn_ref, bf_ref, gf_ref, blf_ref,
    gr_ref, blr_ref, Wr1_ref, br1_ref, Wr2_ref, br2_ref,
    We1_ref, be1_ref, We2_ref, be2_ref,
    W1h_ref, b1h_ref, W2h_ref, b2h_ref,
    heads_ref, lb_ref, ent_ref,
    fused_s, w_s, refined_s, usage_s, ent_s,
):
    m = pl.program_id(0)
    ep = pl.program_id(1)

    @pl.when(ep == 0)
    def _prologue():
        x = x_ref[...]
        num = jax.nn.relu(_ln(jnp.dot(x, Wn_ref[...]) + bn_ref[...],
                              gn_ref[...], bln_ref[...]))
        fused = jax.nn.relu(_ln(jnp.dot(num, Wfn_ref[...]) + bf_ref[...],
                                gf_ref[...], blf_ref[...]))
        fused_s[...] = fused.astype(jnp.bfloat16)
        h = _ln(fused, gr_ref[...], blr_ref[...])
        h1 = jax.nn.relu(jnp.dot(h, Wr1_ref[...]) + br1_ref[...])
        logits = jnp.dot(h1, Wr2_ref[...]) + br2_ref[...]
        w = jax.nn.softmax(logits, axis=-1)
        w_s[...] = w

        @pl.when(m == 0)
        def _init_scalars():
            usage_s[...] = jnp.zeros_like(usage_s)
            ent_s[0, 0] = 0.0

        usage_s[...] += jnp.sum(w, axis=0, keepdims=True)
        ent_s[0, 0] += jnp.sum(w * jnp.log(w + 1e-12))
        refined_s[...] = jnp.dot(w, be2_ref[...])

    fused = fused_s[...].astype(jnp.float32)
    w = w_s[...]
    lane = jax.lax.broadcasted_iota(jnp.int32, (1, E), 1)
    acc = refined_s[...]
    for j in range(E_PER):
        e_abs = ep * E_PER + j
        eh = jax.nn.relu(jnp.dot(fused, We1_ref[j]) + be1_ref[j])
        col = jnp.sum(jnp.where(lane == e_abs, w, 0.0), axis=-1, keepdims=True)
        acc = acc + jnp.dot(eh * col, We2_ref[j])
    refined_s[...] = acc

    @pl.when(ep == E_STEPS - 1)
    def _epilogue():
        r = refined_s[...]
        h1h = jax.nn.relu(jnp.dot(r, W1h_ref[...]) + b1h_ref[...])
        heads_ref[...] = jnp.dot(h1h, W2h_ref[...]) + b2h_ref[...]

        @pl.when(m == M_TILES - 1)
        def _scalars():
            mu = usage_s[...] / B
            lb = jnp.mean((mu - 1.0 / E) ** 2)
            lb_ref[...] = jnp.full((1, 1), lb, jnp.float32)
            ent_ref[...] = jnp.full((1, 1), -ent_s[0, 0] / B, jnp.float32)


def kernel(numeric_features, Wn, bn, gn, bln, Wf, bf, gf, blf, gr, blr,
           Wr1, br1, Wr2, br2, We1, be1, We2, be2, Wt1, bt1, Wt2, bt2,
           Wi1, bi1, Wi2, bi2, Wc1, bc1, Wc2, bc2):
    Wfn = Wf[TEXT_DIM:, :]
    row = lambda v: v.reshape(1, -1)
    N_HEADS = N_TRAITS + 3 + N_CHAR
    W1h = jnp.concatenate([Wt1, Wi1, Wc1], axis=1)
    b1h = jnp.concatenate([bt1, bi1, bc1])
    W2h = jnp.zeros((3 * 256, N_HEADS), jnp.float32)
    W2h = W2h.at[0:256, 0:N_TRAITS].set(Wt2)
    W2h = W2h.at[256:512, N_TRAITS:N_TRAITS + 3].set(Wi2)
    W2h = W2h.at[512:768, N_TRAITS + 3:N_HEADS].set(Wc2)
    b2h = jnp.concatenate([bt2, bi2, bc2])

    const = lambda *_: (0, 0)
    by_m = lambda m, ep: (m, 0)
    by_e3 = lambda m, ep: (ep, 0, 0)

    grid = (M_TILES, E_STEPS)
    out = pl.pallas_call(
        _moe_kernel,
        grid=grid,
        in_specs=[
            pl.BlockSpec((M_TILE, D_NUM), by_m),
            pl.BlockSpec((D_NUM, NUM_HID), const),
            pl.BlockSpec((1, NUM_HID), const),
            pl.BlockSpec((1, NUM_HID), const),
            pl.BlockSpec((1, NUM_HID), const),
            pl.BlockSpec((NUM_HID, FUSION), const),
            pl.BlockSpec((1, FUSION), const),
            pl.BlockSpec((1, FUSION), const),
            pl.BlockSpec((1, FUSION), const),
            pl.BlockSpec((1, FUSION), const),
            pl.BlockSpec((1, FUSION), const),
            pl.BlockSpec((FUSION, FUSION // 2), const),
            pl.BlockSpec((1, FUSION // 2), const),
            pl.BlockSpec((FUSION // 2, E), const),
            pl.BlockSpec((1, E), const),
            pl.BlockSpec((E_PER, FUSION, FUSION), by_e3),
            pl.BlockSpec((E_PER, 1, FUSION), by_e3),
            pl.BlockSpec((E_PER, FUSION, FUSION), by_e3),
            pl.BlockSpec((E, FUSION), const),
            pl.BlockSpec((FUSION, 3 * 256), const),
            pl.BlockSpec((1, 3 * 256), const),
            pl.BlockSpec((3 * 256, N_HEADS), const),
            pl.BlockSpec((1, N_HEADS), const),
        ],
        out_specs=[
            pl.BlockSpec((M_TILE, N_HEADS), by_m),
            pl.BlockSpec((1, 1), const),
            pl.BlockSpec((1, 1), const),
        ],
        out_shape=[
            jax.ShapeDtypeStruct((B, N_HEADS), jnp.float32),
            jax.ShapeDtypeStruct((1, 1), jnp.float32),
            jax.ShapeDtypeStruct((1, 1), jnp.float32),
        ],
        scratch_shapes=[
            pltpu.VMEM((M_TILE, FUSION), jnp.bfloat16),
            pltpu.VMEM((M_TILE, E), jnp.float32),
            pltpu.VMEM((M_TILE, FUSION), jnp.float32),
            pltpu.VMEM((1, E), jnp.float32),
            pltpu.SMEM((1, 1), jnp.float32),
        ],
        compiler_params=pltpu.CompilerParams(
            dimension_semantics=("arbitrary", "arbitrary"),
            vmem_limit_bytes=63 * 1024 * 1024,
        ),
    )(
        numeric_features, Wn, row(bn), row(gn), row(bln), Wfn, row(bf),
        row(gf), row(blf), row(gr), row(blr), Wr1, row(br1), Wr2, row(br2),
        We1, be1.reshape(E, 1, FUSION), We2, be2,
        W1h, row(b1h), W2h, row(b2h),
    )
    heads, lb, ent = out
    trait = heads[:, :N_TRAITS]
    irt = heads[:, N_TRAITS:N_TRAITS + 3]
    char = heads[:, N_TRAITS + 3:]
    return trait, irt, char, lb.reshape(()), ent.reshape(())
